# baseline (speedup 1.0000x reference)
import functools

import jax
import jax.numpy as jnp
from jax import lax
from jax.experimental import pallas as pl
from jax.experimental.pallas import tpu as pltpu
from jax.experimental.pallas import tpu_sc as plsc

LANES = 16
NUM_WORKERS = 32
CHUNK = 16


def _tree_sum(vals):
    vals = list(vals)
    while len(vals) > 1:
        nxt = [vals[i] + vals[i + 1] for i in range(0, len(vals) - 1, 2)]
        if len(vals) % 2:
            nxt.append(vals[-1])
        vals = nxt
    return vals[0]


def _make_sc_kernel(B, L, D, V):
    assert B % NUM_WORKERS == 0 and D % LANES == 0
    ipw = B // NUM_WORKERS
    nch = D // LANES
    inv_l2 = 1.0 / float(L * L)
    n_flat = ipw * L
    assert n_flat % (2 * CHUNK) == 0
    n_pairs = n_flat // (2 * CHUNK)

    mesh = plsc.VectorSubcoreMesh(core_axis_name="c", subcore_axis_name="s")

    @functools.partial(
        pl.kernel,
        out_type=jax.ShapeDtypeStruct((B,), jnp.float32),
        mesh=mesh,
        compiler_params=pltpu.CompilerParams(needs_layout_passes=False),
        scratch_types=[
            pltpu.VMEM((n_flat,), jnp.int32),
            pltpu.VMEM((n_flat,), jnp.int32),
            pltpu.VMEM((CHUNK, D), jnp.float32),
            pltpu.VMEM((CHUNK, D), jnp.float32),
            pltpu.VMEM((CHUNK, D), jnp.float32),
            pltpu.VMEM((CHUNK, D), jnp.float32),
            pltpu.VMEM((ipw + 1, D), jnp.float32),
            pltpu.VMEM((ipw + 1, D), jnp.float32),
            pltpu.VMEM((ipw * LANES,), jnp.float32),
            pltpu.VMEM((ipw,), jnp.float32),
            pltpu.SemaphoreType.DMA,
            pltpu.SemaphoreType.DMA,
            pltpu.SemaphoreType.DMA,
            pltpu.SemaphoreType.DMA,
        ],
    )
    def sc_kernel(text_hbm, code_hbm, wt_hbm, wc_hbm, out_hbm,
                  tidx, cidx, bt0, bt1, bc0, bc1, acc_t, acc_c,
                  partials, outv, st0, st1, sc0, sc1):
        wid = lax.axis_index("s") * 2 + lax.axis_index("c")
        base = wid * n_flat
        pltpu.sync_copy(text_hbm.at[pl.ds(base, n_flat)], tidx)
        pltpu.sync_copy(code_hbm.at[pl.ds(base, n_flat)], cidx)

        zeros = jnp.zeros((LANES,), jnp.float32)

        @pl.loop(0, ipw + 1)
        def _zero(i):
            @pl.loop(0, nch)
            def _col(j):
                col = pl.ds(j * LANES, LANES)
                acc_t[i, col] = zeros
                acc_c[i, col] = zeros

        def reduce_chunk(buf, acc, q):
            row0 = q * CHUNK
            item0 = row0 // L
            bnd = (item0 + 1) * L - row0
            masks = [jnp.zeros((LANES,), jnp.float32) +
                     jnp.where(r < bnd, 1.0, 0.0)
                     for r in range(CHUNK)]

            @pl.loop(0, nch)
            def _col(j):
                col = pl.ds(j * LANES, LANES)
                xs = [buf[r, col] for r in range(CHUNK)]
                sa = _tree_sum([xs[r] * masks[r] for r in range(CHUNK)])
                tot = _tree_sum(xs)
                acc[item0, col] = acc[item0, col] + sa
                acc[item0 + 1, col] = acc[item0 + 1, col] + (tot - sa)

        @pl.loop(0, n_pairs)
        def _pair(p):
            q0 = 2 * p
            q1 = q0 + 1
            o0 = pl.ds(q0 * CHUNK, CHUNK)
            o1 = pl.ds(q1 * CHUNK, CHUNK)
            cps = [
                (pltpu.async_copy(wt_hbm.at[tidx.at[o0]], bt0, st0),
                 bt0, acc_t, q0),
                (pltpu.async_copy(wc_hbm.at[cidx.at[o0]], bc0, sc0),
                 bc0, acc_c, q0),
                (pltpu.async_copy(wt_hbm.at[tidx.at[o1]], bt1, st1),
                 bt1, acc_t, q1),
                (pltpu.async_copy(wc_hbm.at[cidx.at[o1]], bc1, sc1),
                 bc1, acc_c, q1),
            ]
            for cp, buf, acc, q in cps:
                cp.wait()
                reduce_chunk(buf, acc, q)

        @pl.loop(0, ipw)
        def _dot(i):
            def col_body(j, dot_acc):
                col = pl.ds(j * LANES, LANES)
                return dot_acc + acc_t[i, col] * acc_c[i, col]

            dot_acc = lax.fori_loop(
                0, nch, col_body, jnp.zeros((LANES,), jnp.float32))
            partials[pl.ds(i * LANES, LANES)] = dot_acc

        lane = lax.iota(jnp.int32, LANES)
        for g in range(ipw // LANES):
            row_base = g * (LANES * LANES) + lane * LANES
            acc = [plsc.load_gather(partials, [row_base + c]) for c in range(4)]
            for c in range(4, LANES):
                acc[c % 4] = acc[c % 4] + plsc.load_gather(
                    partials, [row_base + c])
            dots = (acc[0] + acc[1]) + (acc[2] + acc[3])
            outv[pl.ds(g * LANES, LANES)] = (
                1.0 / (1.0 + jnp.exp(-dots * inv_l2)))

        pltpu.sync_copy(outv, out_hbm.at[pl.ds(wid * ipw, ipw)])

    return sc_kernel


def kernel(text, code, W_text, W_code):
    B, L = text.shape
    V, D = W_text.shape
    text_flat = text.astype(jnp.int32).reshape(B * L)
    code_flat = code.astype(jnp.int32).reshape(B * L)
    fn = _make_sc_kernel(B, L, D, V)
    return fn(text_flat, code_flat, W_text, W_code)


# speedup vs baseline: 1.0884x; 1.0884x over previous
import functools

import jax
import jax.numpy as jnp
from jax import lax
from jax.experimental import pallas as pl
from jax.experimental.pallas import tpu as pltpu
from jax.experimental.pallas import tpu_sc as plsc

LANES = 16
NUM_WORKERS = 32
CHUNK = 64


def _tree_sum(vals):
    vals = list(vals)
    while len(vals) > 1:
        nxt = [vals[i] + vals[i + 1] for i in range(0, len(vals) - 1, 2)]
        if len(vals) % 2:
            nxt.append(vals[-1])
        vals = nxt
    return vals[0]


def _make_sc_kernel(B, L, D, V):
    assert B % NUM_WORKERS == 0 and D % LANES == 0
    ipw = B // NUM_WORKERS
    nch = D // LANES
    inv_l2 = 1.0 / float(L * L)
    n_flat = ipw * L
    assert n_flat % CHUNK == 0
    n_chunks = n_flat // CHUNK
    n_pairs = n_chunks // 2
    has_tail = n_chunks % 2 == 1

    mesh = plsc.VectorSubcoreMesh(core_axis_name="c", subcore_axis_name="s")

    @functools.partial(
        pl.kernel,
        out_type=jax.ShapeDtypeStruct((B,), jnp.float32),
        mesh=mesh,
        compiler_params=pltpu.CompilerParams(needs_layout_passes=False),
        scratch_types=[
            pltpu.VMEM((n_flat,), jnp.int32),
            pltpu.VMEM((n_flat,), jnp.int32),
            pltpu.VMEM((CHUNK, D), jnp.float32),
            pltpu.VMEM((CHUNK, D), jnp.float32),
            pltpu.VMEM(((ipw + 2) * D,), jnp.float32),
            pltpu.VMEM(((ipw + 2) * LANES,), jnp.float32),
            pltpu.VMEM((ipw,), jnp.float32),
            pltpu.SemaphoreType.DMA,
            pltpu.SemaphoreType.DMA,
        ],
    )
    def sc_kernel(text_hbm, code_hbm, wt_hbm, wc_hbm, out_hbm,
                  tidx, cidx, buf0, buf1, acc_t, partials, outv,
                  sem0, sem1):
        wid = lax.axis_index("s") * 2 + lax.axis_index("c")
        base = wid * n_flat
        pltpu.sync_copy(text_hbm.at[pl.ds(base, n_flat)], tidx)
        pltpu.sync_copy(code_hbm.at[pl.ds(base, n_flat)], cidx)

        zeros = jnp.zeros((LANES,), jnp.float32)

        @pl.loop(0, ipw + 2)
        def _zero(i):
            partials[pl.ds(i * LANES, LANES)] = zeros

            @pl.loop(0, nch)
            def _col(j):
                acc_t[pl.ds(i * D + j * LANES, LANES)] = zeros

        def chunk_geometry(q):
            row0 = q * CHUNK
            item0 = row0 // L
            bnd1 = (item0 + 1) * L - row0
            bnd2 = bnd1 + L
            return item0, bnd1, bnd2

        n_lo = min(CHUNK, L)

        def masked_sums(buf, col, bnd1, bnd2):
            sa_p, base_p, hisel_p, hiall_p = [], [], [], []
            for blk in range(0, CHUNK, LANES):
                xs = [buf[blk + r, col]
                      for r in range(min(LANES, CHUNK - blk))]
                lo = [x for r, x in enumerate(xs) if blk + r < n_lo]
                hi = [x for r, x in enumerate(xs) if blk + r >= n_lo]
                if lo:
                    sa_p.append(_tree_sum(
                        [jnp.where(blk + r < bnd1, x, 0.0)
                         for r, x in enumerate(xs) if blk + r < n_lo]))
                    base_p.append(_tree_sum(lo))
                if hi:
                    hisel_p.append(_tree_sum(
                        [jnp.where(blk + r < bnd2, x, 0.0)
                         for r, x in enumerate(xs) if blk + r >= n_lo]))
                    hiall_p.append(_tree_sum(hi))
            sa = _tree_sum(sa_p)
            base = _tree_sum(base_p)
            hisel = _tree_sum(hisel_p) if hisel_p else None
            hiall = _tree_sum(hiall_p) if hiall_p else None
            sab = base + hisel if hisel is not None else base
            tot = base + hiall if hiall is not None else base
            return sa, sab, tot

        def reduce_text_chunk(buf, q):
            item0, bnd1, bnd2 = chunk_geometry(q)

            @pl.loop(0, nch)
            def _col(j):
                col = pl.ds(j * LANES, LANES)
                sa, sab, tot = masked_sums(buf, col, bnd1, bnd2)
                a0 = pl.ds(item0 * D + j * LANES, LANES)
                a1 = pl.ds((item0 + 1) * D + j * LANES, LANES)
                a2 = pl.ds((item0 + 2) * D + j * LANES, LANES)
                acc_t[a0] = acc_t[a0] + sa
                acc_t[a1] = acc_t[a1] + (sab - sa)
                acc_t[a2] = acc_t[a2] + (tot - sab)

        def reduce_code_chunk(buf, q):
            item0, bnd1, bnd2 = chunk_geometry(q)

            def col_body(j, carry):
                d0, d1, d2 = carry
                col = pl.ds(j * LANES, LANES)
                sa, sab, tot = masked_sums(buf, col, bnd1, bnd2)
                d0 = d0 + acc_t[pl.ds(item0 * D + j * LANES, LANES)] * sa
                d1 = d1 + acc_t[pl.ds((item0 + 1) * D + j * LANES,
                                      LANES)] * (sab - sa)
                d2 = d2 + acc_t[pl.ds((item0 + 2) * D + j * LANES,
                                      LANES)] * (tot - sab)
                return d0, d1, d2

            d0, d1, d2 = lax.fori_loop(
                0, nch, col_body, (zeros, zeros, zeros))
            for off, d in ((0, d0), (1, d1), (2, d2)):
                sl = pl.ds((item0 + off) * LANES, LANES)
                partials[sl] = partials[sl] + d

        def run_phase(idx_ref, tab_hbm, reduce_chunk):
            @pl.loop(0, n_pairs)
            def _pair(p):
                q0 = 2 * p
                q1 = q0 + 1
                cp0 = pltpu.async_copy(
                    tab_hbm.at[idx_ref.at[pl.ds(q0 * CHUNK, CHUNK)]],
                    buf0, sem0)
                cp1 = pltpu.async_copy(
                    tab_hbm.at[idx_ref.at[pl.ds(q1 * CHUNK, CHUNK)]],
                    buf1, sem1)
                cp0.wait()
                reduce_chunk(buf0, q0)
                cp1.wait()
                reduce_chunk(buf1, q1)

            if has_tail:
                q = n_chunks - 1
                pltpu.async_copy(
                    tab_hbm.at[idx_ref.at[pl.ds(q * CHUNK, CHUNK)]],
                    buf0, sem0).wait()
                reduce_chunk(buf0, q)

        run_phase(tidx, wt_hbm, reduce_text_chunk)
        run_phase(cidx, wc_hbm, reduce_code_chunk)

        lane = lax.iota(jnp.int32, LANES)
        for g in range(ipw // LANES):
            row_base = g * (LANES * LANES) + lane * LANES
            acc = [plsc.load_gather(partials, [row_base + c]) for c in range(4)]
            for c in range(4, LANES):
                acc[c % 4] = acc[c % 4] + plsc.load_gather(
                    partials, [row_base + c])
            dots = (acc[0] + acc[1]) + (acc[2] + acc[3])
            outv[pl.ds(g * LANES, LANES)] = (
                1.0 / (1.0 + jnp.exp(-dots * inv_l2)))

        pltpu.sync_copy(outv, out_hbm.at[pl.ds(wid * ipw, ipw)])

    return sc_kernel


def kernel(text, code, W_text, W_code):
    B, L = text.shape
    V, D = W_text.shape
    text_flat = text.astype(jnp.int32).reshape(B * L)
    code_flat = code.astype(jnp.int32).reshape(B * L)
    fn = _make_sc_kernel(B, L, D, V)
    return fn(text_flat, code_flat, W_text, W_code)


# speedup vs baseline: 1.3663x; 1.2554x over previous
import functools
import math

import jax
import jax.numpy as jnp
from jax import lax
from jax.experimental import pallas as pl
from jax.experimental.pallas import tpu as pltpu
from jax.experimental.pallas import tpu_sc as plsc

LANES = 16
NUM_WORKERS = 32
CHUNK = 40


def _make_sc_kernel(B, L, D, V):
    assert B % NUM_WORKERS == 0 and D % LANES == 0
    ipw = B // NUM_WORKERS
    nch = D // LANES
    inv_l2 = 1.0 / float(L * L)
    n_flat = ipw * L
    assert n_flat % CHUNK == 0
    period = math.lcm(CHUNK, L)
    chunks_per_group = period // CHUNK
    items_per_group = period // L
    n_groups = n_flat // period
    assert n_groups * period == n_flat

    segs = []
    for k in range(chunks_per_group):
        lo, hi = k * CHUNK, (k + 1) * CHUNK
        cur = []
        for li in range(items_per_group):
            a, b = max(lo, li * L), min(hi, (li + 1) * L)
            if a < b:
                cur.append((li, a - lo, b - lo, a == li * L))
        segs.append(cur)

    mesh = plsc.VectorSubcoreMesh(core_axis_name="c", subcore_axis_name="s")

    @functools.partial(
        pl.kernel,
        out_type=jax.ShapeDtypeStruct((B,), jnp.float32),
        mesh=mesh,
        compiler_params=pltpu.CompilerParams(needs_layout_passes=False),
        scratch_types=[
            pltpu.VMEM((n_flat,), jnp.int32),
            pltpu.VMEM((n_flat,), jnp.int32),
            pltpu.VMEM((CHUNK, D), jnp.float32),
            pltpu.VMEM((CHUNK, D), jnp.float32),
            pltpu.VMEM((ipw, D), jnp.float32),
            pltpu.VMEM((ipw, D), jnp.float32),
            pltpu.VMEM((ipw * LANES,), jnp.float32),
            pltpu.VMEM((ipw,), jnp.float32),
            pltpu.SemaphoreType.DMA,
            pltpu.SemaphoreType.DMA,
        ],
    )
    def sc_kernel(text_hbm, code_hbm, wt_hbm, wc_hbm, out_hbm,
                  tidx, cidx, buf_t, buf_c, acc_t, acc_c, partials, outv,
                  sem_t, sem_c):
        wid = lax.axis_index("s") * 2 + lax.axis_index("c")
        base = wid * n_flat
        pltpu.sync_copy(text_hbm.at[pl.ds(base, n_flat)], tidx)
        pltpu.sync_copy(code_hbm.at[pl.ds(base, n_flat)], cidx)

        def accum_chunk(buf, acc, k, item0):
            for li, r0, r1, first in segs[k]:
                item = item0 + li
                n = r1 - r0

                @pl.loop(0, nch)
                def _col(j):
                    col = pl.ds(j * LANES, LANES)
                    s = [buf[r0 + r, col] for r in range(min(4, n))]
                    for r in range(4, n):
                        s[r % 4] = s[r % 4] + buf[r0 + r, col]
                    while len(s) > 1:
                        s = [s[0] + s[1]] + s[2:]
                    if first:
                        acc[item, col] = s[0]
                    else:
                        acc[item, col] = acc[item, col] + s[0]

        @pl.loop(0, n_groups)
        def _group(g):
            item0 = g * items_per_group
            for k in range(chunks_per_group):
                off = pl.ds(g * period + k * CHUNK, CHUNK)
                cp_t = pltpu.async_copy(wt_hbm.at[tidx.at[off]], buf_t, sem_t)
                cp_c = pltpu.async_copy(wc_hbm.at[cidx.at[off]], buf_c, sem_c)
                cp_t.wait()
                accum_chunk(buf_t, acc_t, k, item0)
                cp_c.wait()
                accum_chunk(buf_c, acc_c, k, item0)

        @pl.loop(0, ipw)
        def _dot(i):
            def col_body(j, dot_acc):
                col = pl.ds(j * LANES, LANES)
                return dot_acc + acc_t[i, col] * acc_c[i, col]

            dot_acc = lax.fori_loop(
                0, nch, col_body, jnp.zeros((LANES,), jnp.float32))
            partials[pl.ds(i * LANES, LANES)] = dot_acc

        lane = lax.iota(jnp.int32, LANES)
        for g in range(ipw // LANES):
            row_base = g * (LANES * LANES) + lane * LANES
            acc = [plsc.load_gather(partials, [row_base + c]) for c in range(4)]
            for c in range(4, LANES):
                acc[c % 4] = acc[c % 4] + plsc.load_gather(
                    partials, [row_base + c])
            dots = (acc[0] + acc[1]) + (acc[2] + acc[3])
            outv[pl.ds(g * LANES, LANES)] = (
                1.0 / (1.0 + jnp.exp(-dots * inv_l2)))

        pltpu.sync_copy(outv, out_hbm.at[pl.ds(wid * ipw, ipw)])

    return sc_kernel


def kernel(text, code, W_text, W_code):
    B, L = text.shape
    V, D = W_text.shape
    text_flat = text.astype(jnp.int32).reshape(B * L)
    code_flat = code.astype(jnp.int32).reshape(B * L)
    fn = _make_sc_kernel(B, L, D, V)
    return fn(text_flat, code_flat, W_text, W_code)


# speedup vs baseline: 2.3597x; 1.7270x over previous
import functools
import math

import jax
import jax.numpy as jnp
from jax import lax
from jax.experimental import pallas as pl
from jax.experimental.pallas import tpu as pltpu
from jax.experimental.pallas import tpu_sc as plsc

LANES = 16
NUM_WORKERS = 32
CHUNK = 40
LAG = 3


def _make_sc_kernel(B, L, D, V):
    assert B % NUM_WORKERS == 0 and D % LANES == 0
    ipw = B // NUM_WORKERS
    nch = D // LANES
    inv_l2 = 1.0 / float(L * L)
    n_flat = ipw * L
    assert n_flat % CHUNK == 0
    n_chunks = n_flat // CHUNK
    period = math.lcm(CHUNK, L)
    cpg = period // CHUNK
    ipg = period // L
    n_steps = n_chunks + LAG
    body = 2 * cpg
    assert (n_steps - LAG) % body == 0
    n_bodies = (n_steps - LAG) // body

    segs = []
    for kk in range(cpg):
        lo, hi = kk * CHUNK, (kk + 1) * CHUNK
        cur = []
        for li in range(ipg):
            a, b = max(lo, li * L), min(hi, (li + 1) * L)
            if a < b:
                cur.append((li, a - lo, b - lo, a == li * L))
        segs.append(cur)

    mesh = plsc.VectorSubcoreMesh(core_axis_name="c", subcore_axis_name="s")

    @functools.partial(
        pl.kernel,
        out_type=jax.ShapeDtypeStruct((B,), jnp.float32),
        mesh=mesh,
        compiler_params=pltpu.CompilerParams(needs_layout_passes=False),
        scratch_types=[
            pltpu.VMEM((n_flat,), jnp.int32),
            pltpu.VMEM((n_flat,), jnp.int32),
            pltpu.VMEM((CHUNK, D), jnp.float32),
            pltpu.VMEM((CHUNK, D), jnp.float32),
            pltpu.VMEM((CHUNK, D), jnp.float32),
            pltpu.VMEM((ipw, D), jnp.float32),
            pltpu.VMEM((ipw * LANES,), jnp.float32),
            pltpu.VMEM((ipw,), jnp.float32),
            pltpu.SemaphoreType.DMA,
            pltpu.SemaphoreType.DMA,
            pltpu.SemaphoreType.DMA,
        ],
    )
    def sc_kernel(text_hbm, code_hbm, wt_hbm, wc_hbm, out_hbm,
                  tidx, cidx, bt0, bt1, bc, acc_t, partials, outv,
                  st0, st1, sc_sem):
        wid = lax.axis_index("s") * 2 + lax.axis_index("c")
        base = wid * n_flat
        pltpu.sync_copy(text_hbm.at[pl.ds(base, n_flat)], tidx)
        pltpu.sync_copy(code_hbm.at[pl.ds(base, n_flat)], cidx)

        zeros = jnp.zeros((LANES,), jnp.float32)

        @pl.loop(0, ipw)
        def _zero(i):
            partials[pl.ds(i * LANES, LANES)] = zeros

        tslots = ((bt0, st0), (bt1, st1))

        def t_copy(q, slot):
            buf, sem = tslots[slot]
            return pltpu.make_async_copy(
                wt_hbm.at[tidx.at[pl.ds(q * CHUNK, CHUNK)]], buf, sem)

        def c_copy(q):
            return pltpu.make_async_copy(
                wc_hbm.at[cidx.at[pl.ds(q * CHUNK, CHUNK)]], bc, sc_sem)

        def item_base(k, kk):
            return (k - kk) // cpg * ipg

        def compute_t(buf, k, kk):
            item0 = item_base(k, kk)
            for li, r0, r1, first in segs[kk]:
                item = item0 + li
                n = r1 - r0

                @pl.loop(0, nch)
                def _col(j):
                    col = pl.ds(j * LANES, LANES)
                    s = [buf[r0 + r, col] for r in range(min(4, n))]
                    for r in range(4, n):
                        s[r % 4] = s[r % 4] + buf[r0 + r, col]
                    while len(s) > 1:
                        s = [s[0] + s[1]] + s[2:]
                    if first:
                        acc_t[item, col] = s[0]
                    else:
                        acc_t[item, col] = acc_t[item, col] + s[0]

        def compute_c(k, kk):
            item0 = item_base(k, kk)
            for li, r0, r1, _ in segs[kk]:
                item = item0 + li
                n = r1 - r0

                def col_body(j, d):
                    col = pl.ds(j * LANES, LANES)
                    s = [bc[r0 + r, col] for r in range(min(4, n))]
                    for r in range(4, n):
                        s[r % 4] = s[r % 4] + bc[r0 + r, col]
                    while len(s) > 1:
                        s = [s[0] + s[1]] + s[2:]
                    return d + acc_t[item, col] * s[0]

                d = lax.fori_loop(0, nch, col_body, zeros)
                sl = pl.ds(item * LANES, LANES)
                partials[sl] = partials[sl] + d

        t_copy(0, 0).start()
        t_copy(1, 1).start()
        c_copy(0).start()
        for q in range(LAG):
            slot = q % 2
            t_copy(q, slot).wait()
            compute_t(tslots[slot][0], q, q % cpg)
            t_copy(q + 2, slot).start()

        @pl.loop(0, n_bodies)
        def _pipe(p):
            for j in range(body):
                q = LAG + p * body + j
                qs = LAG + j
                slot = qs % 2

                @pl.when(q <= n_chunks - 1)
                def _twork():
                    t_copy(q, slot).wait()
                    compute_t(tslots[slot][0], q, qs % cpg)

                @pl.when(q + 2 <= n_chunks - 1)
                def _tissue():
                    t_copy(q + 2, slot).start()

                kc = q - LAG
                c_copy(kc).wait()
                compute_c(kc, j % cpg)

                @pl.when(kc + 1 <= n_chunks - 1)
                def _cissue():
                    c_copy(kc + 1).start()

        lane = lax.iota(jnp.int32, LANES)
        for g in range(ipw // LANES):
            row_base = g * (LANES * LANES) + lane * LANES
            acc = [plsc.load_gather(partials, [row_base + c]) for c in range(4)]
            for c in range(4, LANES):
                acc[c % 4] = acc[c % 4] + plsc.load_gather(
                    partials, [row_base + c])
            dots = (acc[0] + acc[1]) + (acc[2] + acc[3])
            outv[pl.ds(g * LANES, LANES)] = (
                1.0 / (1.0 + jnp.exp(-dots * inv_l2)))

        pltpu.sync_copy(outv, out_hbm.at[pl.ds(wid * ipw, ipw)])

    return sc_kernel


def kernel(text, code, W_text, W_code):
    B, L = text.shape
    V, D = W_text.shape
    text_flat = text.astype(jnp.int32).reshape(B * L)
    code_flat = code.astype(jnp.int32).reshape(B * L)
    fn = _make_sc_kernel(B, L, D, V)
    return fn(text_flat, code_flat, W_text, W_code)
